# Initial kernel scaffold; baseline (speedup 1.0000x reference)
import jax
import jax.numpy as jnp
from jax.experimental import pallas as pl


def kernel(x, edge_index, edge_attr, w1, b1, ew1, eb1, w2, b2, ew2, eb2, w3, b3, ew3, eb3, w4, b4, ew4, eb4, w5, b5, ew5, eb5):
    raise NotImplementedError("write your pallas kernel here")


# speedup vs baseline: 7.8296x; 7.8296x over previous
import functools

import jax
import jax.numpy as jnp
from jax import lax
from jax.experimental import pallas as pl
from jax.experimental.pallas import tpu as pltpu
from jax.experimental.pallas import tpu_sc as plsc

NC = 2
NS = 16
NW = NC * NS


def _sc_mesh():
    return plsc.VectorSubcoreMesh(core_axis_name="c", subcore_axis_name="s")


def _sc_degree(rows3, ones128, z128, n):
    nch, k = rows3.shape[1], rows3.shape[2]
    rps = n // NS

    @functools.partial(
        pl.kernel,
        mesh=_sc_mesh(),
        out_type=jax.ShapeDtypeStruct((NC * n, 128), jnp.float32),
        scratch_types=[
            pltpu.VMEM_SHARED((n, 128), jnp.float32),
            pltpu.VMEM((k,), jnp.int32),
            pltpu.VMEM((k, 128), jnp.float32),
        ],
    )
    def deg_kernel(rows_hbm, ones_hbm, z_hbm, out_hbm, acc_sh, ri_v, ones_v):
        cid = lax.axis_index("c")
        sid = lax.axis_index("s")
        wid = sid * NC + cid
        pltpu.sync_copy(z_hbm, acc_sh.at[pl.ds(sid * rps, rps)])
        pltpu.sync_copy(ones_hbm, ones_v)
        plsc.subcore_barrier()

        def chunk(i, carry):
            pltpu.sync_copy(rows_hbm.at[wid, i], ri_v)
            pltpu.sync_copy(ones_v, acc_sh.at[ri_v], add=True)
            return carry

        lax.fori_loop(0, nch, chunk, 0)
        plsc.subcore_barrier()
        pltpu.sync_copy(
            acc_sh.at[pl.ds(sid * rps, rps)],
            out_hbm.at[pl.ds(cid * n + sid * rps, rps)],
        )

    return deg_kernel(rows3, ones128, z128)


def _sc_sea(rows3, cols3, ea3, dis128, z128, n):
    nch, k = rows3.shape[1], rows3.shape[2]
    rps = n // NS

    @functools.partial(
        pl.kernel,
        mesh=_sc_mesh(),
        out_type=jax.ShapeDtypeStruct((NC * n, 128), jnp.float32),
        scratch_types=[
            pltpu.VMEM_SHARED((n, 128), jnp.float32),
            pltpu.VMEM((k,), jnp.int32),
            pltpu.VMEM((k,), jnp.int32),
            pltpu.VMEM((k, 128), jnp.float32),
            pltpu.VMEM((k, 16), jnp.float32),
            pltpu.SemaphoreType.DMA,
        ],
    )
    def sea_kernel(rows_hbm, cols_hbm, ea_hbm, dis_hbm, z_hbm, out_hbm,
                   acc_sh, ri_v, ci_v, g_v, ea_v, sem):
        cid = lax.axis_index("c")
        sid = lax.axis_index("s")
        wid = sid * NC + cid
        pltpu.sync_copy(z_hbm, acc_sh.at[pl.ds(sid * rps, rps)])
        plsc.subcore_barrier()

        def chunk(i, carry):
            pltpu.sync_copy(rows_hbm.at[wid, i], ri_v)
            pltpu.sync_copy(cols_hbm.at[wid, i], ci_v)
            pltpu.async_copy(dis_hbm.at[ri_v], g_v, sem).wait()
            pltpu.sync_copy(ea_hbm.at[wid * nch + i], ea_v)
            for e in range(k):
                g_v[e, 0:16] = g_v[e, 0:16] * ea_v[e, 0:16]
            pltpu.sync_copy(g_v, acc_sh.at[ci_v], add=True)
            return carry

        lax.fori_loop(0, nch, chunk, 0)
        plsc.subcore_barrier()
        pltpu.sync_copy(
            acc_sh.at[pl.ds(sid * rps, rps)],
            out_hbm.at[pl.ds(cid * n + sid * rps, rps)],
        )

    return sea_kernel(rows3, cols3, ea3, dis128, z128)


def _sc_agg(y, rows3, cols3, z128, n, d, np_):
    nch, k = rows3.shape[1], rows3.shape[2]
    rps = np_ // NS

    @functools.partial(
        pl.kernel,
        mesh=_sc_mesh(),
        out_type=jax.ShapeDtypeStruct((NC * np_, d), jnp.float32),
        scratch_types=[
            pltpu.VMEM_SHARED((np_, d), jnp.float32),
            pltpu.VMEM((k,), jnp.int32),
            pltpu.VMEM((k,), jnp.int32),
            pltpu.VMEM((k, d), jnp.float32),
            pltpu.SemaphoreType.DMA,
        ],
    )
    def agg_kernel(y_hbm, rows_hbm, cols_hbm, z_hbm, out_hbm,
                   acc_sh, ri_v, ci_v, rows_v, sem):
        cid = lax.axis_index("c")
        sid = lax.axis_index("s")
        wid = sid * NC + cid
        pltpu.sync_copy(z_hbm, acc_sh.at[pl.ds(sid * rps, rps)])
        plsc.subcore_barrier()

        def chunk(i, carry):
            pltpu.sync_copy(rows_hbm.at[wid, i], ri_v)
            pltpu.sync_copy(cols_hbm.at[wid, i], ci_v)
            pltpu.async_copy(y_hbm.at[ri_v], rows_v, sem).wait()
            pltpu.sync_copy(rows_v, acc_sh.at[ci_v], add=True)
            return carry

        lax.fori_loop(0, nch, chunk, 0)
        plsc.subcore_barrier()
        pltpu.sync_copy(
            acc_sh.at[pl.ds(sid * rps, rps)],
            out_hbm.at[pl.ds(cid * np_ + sid * rps, rps)],
        )

    return agg_kernel(y, rows3, cols3, z128)


def _tc_dis(deg2, n):
    def body(deg_ref, dis_ref, dis128_ref):
        d = deg_ref[0, :, 0:1] + deg_ref[1, :, 0:1] + 1.0
        dis = lax.rsqrt(d)
        dis_ref[...] = dis
        b16 = jnp.broadcast_to(dis, (n, 16))
        z111 = jnp.zeros((n, 111), jnp.float32)
        dis128_ref[...] = jnp.concatenate([b16, dis, z111], axis=1)

    return pl.pallas_call(
        body,
        out_shape=(
            jax.ShapeDtypeStruct((n, 1), jnp.float32),
            jax.ShapeDtypeStruct((n, 128), jnp.float32),
        ),
    )(deg2)


def _tc_e0(sea2, dis1, ewts, ebs3, bs3, n, d, nl):
    def body(sea_ref, dis_ref, ewt_ref, eb_ref, b_ref, out_ref):
        dis = dis_ref[...]
        sea = dis * (sea_ref[0, :, 0:16] + sea_ref[1, :, 0:16])
        ssum = dis * (sea_ref[0, :, 16:17] + sea_ref[1, :, 16:17]) + dis * dis
        term = jnp.dot(sea, ewt_ref[0], preferred_element_type=jnp.float32)
        out_ref[0] = term + ssum * eb_ref[0] + b_ref[0]

    return pl.pallas_call(
        body,
        grid=(nl,),
        in_specs=[
            pl.BlockSpec((2, n, 32), lambda l: (0, 0, 0)),
            pl.BlockSpec((n, 1), lambda l: (0, 0)),
            pl.BlockSpec((1, 16, d), lambda l: (l, 0, 0)),
            pl.BlockSpec((1, 1, d), lambda l: (l, 0, 0)),
            pl.BlockSpec((1, 1, d), lambda l: (l, 0, 0)),
        ],
        out_specs=pl.BlockSpec((1, n, d), lambda l: (l, 0, 0)),
        out_shape=jax.ShapeDtypeStruct((nl, n, d), jnp.float32),
    )(sea2, dis1, ewts, ebs3, bs3)


def _tc_y(h, w, dis1, n, d, br):
    def body(h_ref, w_ref, dis_ref, out_ref):
        out_ref[...] = dis_ref[...] * jnp.dot(
            h_ref[...], w_ref[...], preferred_element_type=jnp.float32)

    return pl.pallas_call(
        body,
        grid=(n // br,),
        in_specs=[
            pl.BlockSpec((br, d), lambda i: (i, 0)),
            pl.BlockSpec((d, d), lambda i: (0, 0)),
            pl.BlockSpec((br, 1), lambda i: (i, 0)),
        ],
        out_specs=pl.BlockSpec((br, d), lambda i: (i, 0)),
        out_shape=jax.ShapeDtypeStruct((n, d), jnp.float32),
    )(h, w, dis1)


def _tc_fused(agg2, y, e0, dis1, w_next, lidx, n, d, br):
    def body(agg_ref, y_ref, e0_ref, dis_ref, w_ref, out_ref):
        dis = dis_ref[...]
        h = dis * (agg_ref[0] + agg_ref[1] + y_ref[...]) + e0_ref[0]
        h = jnp.maximum(h, 0.0)
        out_ref[...] = dis * jnp.dot(
            h, w_ref[...], preferred_element_type=jnp.float32)

    return pl.pallas_call(
        body,
        grid=(n // br,),
        in_specs=[
            pl.BlockSpec((2, br, d), lambda i: (0, i, 0)),
            pl.BlockSpec((br, d), lambda i: (i, 0)),
            pl.BlockSpec((1, br, d), lambda i: (lidx, i, 0)),
            pl.BlockSpec((br, 1), lambda i: (i, 0)),
            pl.BlockSpec((d, d), lambda i: (0, 0)),
        ],
        out_specs=pl.BlockSpec((br, d), lambda i: (i, 0)),
        out_shape=jax.ShapeDtypeStruct((n, d), jnp.float32),
    )(agg2, y, e0, dis1, w_next)


def _tc_final(agg2, y, e0, dis1, lidx, n, d, br):
    def body(agg_ref, y_ref, e0_ref, dis_ref, out_ref):
        dis = dis_ref[...]
        out_ref[...] = dis * (agg_ref[0] + agg_ref[1] + y_ref[...]) + e0_ref[0]

    return pl.pallas_call(
        body,
        grid=(n // br,),
        in_specs=[
            pl.BlockSpec((2, br, d), lambda i: (0, i, 0)),
            pl.BlockSpec((br, d), lambda i: (i, 0)),
            pl.BlockSpec((1, br, d), lambda i: (lidx, i, 0)),
            pl.BlockSpec((br, 1), lambda i: (i, 0)),
        ],
        out_specs=pl.BlockSpec((br, d), lambda i: (i, 0)),
        out_shape=jax.ShapeDtypeStruct((n, d), jnp.float32),
    )(agg2, y, e0, dis1)


def kernel(x, edge_index, edge_attr,
           w1, b1, ew1, eb1,
           w2, b2, ew2, eb2,
           w3, b3, ew3, eb3,
           w4, b4, ew4, eb4,
           w5, b5, ew5, eb5):
    n, d = x.shape
    e = edge_index.shape[1]
    de = edge_attr.shape[1]
    nl = 5
    br = 2000

    epw = e // NW
    k = 100
    nch = epw // k

    rows3 = edge_index[0].reshape(NW, nch, k)
    cols3 = edge_index[1].reshape(NW, nch, k)
    ea3 = edge_attr.reshape(NW * nch, k, de)

    np_ = ((n + 8 * NS - 1) // (8 * NS)) * (8 * NS)
    rps = np_ // NS
    z128 = jnp.zeros((rps, d), jnp.float32)
    ones128 = jnp.concatenate(
        [jnp.ones((k, 1), jnp.float32), jnp.zeros((k, 127), jnp.float32)], axis=1)

    ws = [w1, w2, w3, w4, w5]
    ewts = jnp.stack([ew1.T, ew2.T, ew3.T, ew4.T, ew5.T])
    ebs3 = jnp.stack([eb1, eb2, eb3, eb4, eb5])[:, None, :]
    bs3 = jnp.stack([b1, b2, b3, b4, b5])[:, None, :]

    deg2 = _sc_degree(rows3, ones128, z128, np_).reshape(NC, np_, d)[:, :n, :16]
    dis1, dis128 = _tc_dis(deg2, n)
    sea2 = _sc_sea(rows3, cols3, ea3, dis128, z128, np_).reshape(NC, np_, d)[:, :n, :32]
    e0 = _tc_e0(sea2, dis1, ewts, ebs3, bs3, n, d, nl)

    y = _tc_y(x, ws[0], dis1, n, d, br)
    for l in range(nl):
        agg2 = _sc_agg(y, rows3, cols3, z128, n, d, np_).reshape(NC, np_, d)[:, :n]
        if l < nl - 1:
            y = _tc_fused(agg2, y, e0, dis1, ws[l + 1], l, n, d, br)
        else:
            return _tc_final(agg2, y, e0, dis1, l, n, d, br)


# speedup vs baseline: 12.4195x; 1.5862x over previous
import functools

import jax
import jax.numpy as jnp
from jax import lax
from jax.experimental import pallas as pl
from jax.experimental.pallas import tpu as pltpu
from jax.experimental.pallas import tpu_sc as plsc

NC = 2
NS = 16
NW = NC * NS


def _sc_mesh():
    return plsc.VectorSubcoreMesh(core_axis_name="c", subcore_axis_name="s")


def _sc_degree(rows3, ones128, z128, n):
    nch, k = rows3.shape[1], rows3.shape[2]
    rps = n // NS

    @functools.partial(
        pl.kernel,
        mesh=_sc_mesh(),
        out_type=jax.ShapeDtypeStruct((NC * n, 128), jnp.float32),
        scratch_types=[
            pltpu.VMEM_SHARED((n, 128), jnp.float32),
            pltpu.VMEM((k,), jnp.int32),
            pltpu.VMEM((k, 128), jnp.float32),
        ],
    )
    def deg_kernel(rows_hbm, ones_hbm, z_hbm, out_hbm, acc_sh, ri_v, ones_v):
        cid = lax.axis_index("c")
        sid = lax.axis_index("s")
        wid = sid * NC + cid
        pltpu.sync_copy(z_hbm, acc_sh.at[pl.ds(sid * rps, rps)])
        pltpu.sync_copy(ones_hbm, ones_v)
        plsc.subcore_barrier()

        def chunk(i, carry):
            pltpu.sync_copy(rows_hbm.at[wid, i], ri_v)
            pltpu.sync_copy(ones_v, acc_sh.at[ri_v], add=True)
            return carry

        lax.fori_loop(0, nch, chunk, 0)
        plsc.subcore_barrier()
        pltpu.sync_copy(
            acc_sh.at[pl.ds(sid * rps, rps)],
            out_hbm.at[pl.ds(cid * n + sid * rps, rps)],
        )

    return deg_kernel(rows3, ones128, z128)


def _sc_sea(rows3, cols3, ea3, dis128, z128, n):
    nch, k = rows3.shape[1], rows3.shape[2]
    rps = n // NS

    @functools.partial(
        pl.kernel,
        mesh=_sc_mesh(),
        out_type=jax.ShapeDtypeStruct((NC * n, 128), jnp.float32),
        scratch_types=[
            pltpu.VMEM_SHARED((n, 128), jnp.float32),
            pltpu.VMEM((k,), jnp.int32),
            pltpu.VMEM((k,), jnp.int32),
            pltpu.VMEM((k, 128), jnp.float32),
            pltpu.VMEM((k, 16), jnp.float32),
            pltpu.SemaphoreType.DMA,
        ],
    )
    def sea_kernel(rows_hbm, cols_hbm, ea_hbm, dis_hbm, z_hbm, out_hbm,
                   acc_sh, ri_v, ci_v, g_v, ea_v, sem):
        cid = lax.axis_index("c")
        sid = lax.axis_index("s")
        wid = sid * NC + cid
        pltpu.sync_copy(z_hbm, acc_sh.at[pl.ds(sid * rps, rps)])
        plsc.subcore_barrier()

        def chunk(i, carry):
            pltpu.sync_copy(rows_hbm.at[wid, i], ri_v)
            pltpu.sync_copy(cols_hbm.at[wid, i], ci_v)
            pltpu.async_copy(dis_hbm.at[ri_v], g_v, sem).wait()
            pltpu.sync_copy(ea_hbm.at[wid * nch + i], ea_v)
            for e in range(k):
                g_v[e, 0:16] = g_v[e, 0:16] * ea_v[e, 0:16]
            pltpu.sync_copy(g_v, acc_sh.at[ci_v], add=True)
            return carry

        lax.fori_loop(0, nch, chunk, 0)
        plsc.subcore_barrier()
        pltpu.sync_copy(
            acc_sh.at[pl.ds(sid * rps, rps)],
            out_hbm.at[pl.ds(cid * n + sid * rps, rps)],
        )

    return sea_kernel(rows3, cols3, ea3, dis128, z128)


def _sc_agg(y, packed3, z128, n, d, np_):
    nch, k = packed3.shape[1], packed3.shape[2]
    rps = np_ // NS
    NB = 2

    @functools.partial(
        pl.kernel,
        mesh=_sc_mesh(),
        out_type=jax.ShapeDtypeStruct((NC * np_, d), jnp.float32),
        scratch_types=[
            pltpu.VMEM_SHARED((np_, d), jnp.float32),
            pltpu.VMEM((nch, k), jnp.int32),
            pltpu.VMEM((k,), jnp.int32),
            pltpu.VMEM((k,), jnp.int32),
            pltpu.VMEM((k,), jnp.int32),
            pltpu.VMEM((k,), jnp.int32),
            pltpu.VMEM((k, d), jnp.float32),
            pltpu.VMEM((k, d), jnp.float32),
            pltpu.SemaphoreType.DMA,
            pltpu.SemaphoreType.DMA,
        ],
    )
    def agg_kernel(y_hbm, packed_hbm, z_hbm, out_hbm, acc_sh, pk_v,
                   ri0, ci0, ri1, ci1, rows0, rows1, gsem0, gsem1):
        cid = lax.axis_index("c")
        sid = lax.axis_index("s")
        wid = sid * NC + cid
        pltpu.sync_copy(z_hbm, acc_sh.at[pl.ds(sid * rps, rps)])
        pltpu.sync_copy(packed_hbm.at[wid], pk_v)
        plsc.subcore_barrier()

        ris = [ri0, ri1]
        cis = [ci0, ci1]
        rows = [rows0, rows1]
        gsems = [gsem0, gsem1]

        def unpack(i, b):
            for j in range(k // 16):
                v = pk_v[i, pl.ds(j * 16, 16)]
                ris[b][pl.ds(j * 16, 16)] = jnp.bitwise_and(v, 0xFFFF)
                cis[b][pl.ds(j * 16, 16)] = lax.shift_right_logical(v, 16)

        for b in range(NB):
            unpack(b, b)
            pltpu.async_copy(y_hbm.at[ris[b]], rows[b], gsems[b])

        ngrp = (nch + NB - 1) // NB

        def group(g, carry):
            for b in range(NB):
                i = g * NB + b

                @pl.when(i < nch)
                def _():
                    pltpu.make_async_copy(y_hbm.at[ris[b]], rows[b],
                                          gsems[b]).wait()
                    pltpu.sync_copy(rows[b], acc_sh.at[cis[b]], add=True)

                    @pl.when(i + NB < nch)
                    def _():
                        unpack(i + NB, b)
                        pltpu.async_copy(y_hbm.at[ris[b]], rows[b], gsems[b])
            return carry

        lax.fori_loop(0, ngrp, group, 0)
        plsc.subcore_barrier()
        pltpu.sync_copy(
            acc_sh.at[pl.ds(sid * rps, rps)],
            out_hbm.at[pl.ds(cid * np_ + sid * rps, rps)],
        )

    return agg_kernel(y, packed3, z128)


def _tc_dis(deg2, n):
    def body(deg_ref, dis_ref, dis128_ref):
        d = deg_ref[0, :, 0:1] + deg_ref[1, :, 0:1] + 1.0
        dis = lax.rsqrt(d)
        dis_ref[...] = dis
        b16 = jnp.broadcast_to(dis, (n, 16))
        z111 = jnp.zeros((n, 111), jnp.float32)
        dis128_ref[...] = jnp.concatenate([b16, dis, z111], axis=1)

    return pl.pallas_call(
        body,
        out_shape=(
            jax.ShapeDtypeStruct((n, 1), jnp.float32),
            jax.ShapeDtypeStruct((n, 128), jnp.float32),
        ),
    )(deg2)


def _tc_e0(sea2, dis1, ewts, ebs3, bs3, n, d, nl):
    def body(sea_ref, dis_ref, ewt_ref, eb_ref, b_ref, out_ref):
        dis = dis_ref[...]
        sea = dis * (sea_ref[0, :, 0:16] + sea_ref[1, :, 0:16])
        ssum = dis * (sea_ref[0, :, 16:17] + sea_ref[1, :, 16:17]) + dis * dis
        term = jnp.dot(sea, ewt_ref[0], preferred_element_type=jnp.float32)
        out_ref[0] = term + ssum * eb_ref[0] + b_ref[0]

    return pl.pallas_call(
        body,
        grid=(nl,),
        in_specs=[
            pl.BlockSpec((2, n, 32), lambda l: (0, 0, 0)),
            pl.BlockSpec((n, 1), lambda l: (0, 0)),
            pl.BlockSpec((1, 16, d), lambda l: (l, 0, 0)),
            pl.BlockSpec((1, 1, d), lambda l: (l, 0, 0)),
            pl.BlockSpec((1, 1, d), lambda l: (l, 0, 0)),
        ],
        out_specs=pl.BlockSpec((1, n, d), lambda l: (l, 0, 0)),
        out_shape=jax.ShapeDtypeStruct((nl, n, d), jnp.float32),
    )(sea2, dis1, ewts, ebs3, bs3)


def _tc_y(h, w, dis1, n, d, br):
    def body(h_ref, w_ref, dis_ref, out_ref):
        out_ref[...] = dis_ref[...] * jnp.dot(
            h_ref[...], w_ref[...], preferred_element_type=jnp.float32)

    return pl.pallas_call(
        body,
        grid=(n // br,),
        in_specs=[
            pl.BlockSpec((br, d), lambda i: (i, 0)),
            pl.BlockSpec((d, d), lambda i: (0, 0)),
            pl.BlockSpec((br, 1), lambda i: (i, 0)),
        ],
        out_specs=pl.BlockSpec((br, d), lambda i: (i, 0)),
        out_shape=jax.ShapeDtypeStruct((n, d), jnp.float32),
    )(h, w, dis1)


def _tc_fused(agg2, y, e0, dis1, w_next, lidx, n, d, br):
    def body(agg_ref, y_ref, e0_ref, dis_ref, w_ref, out_ref):
        dis = dis_ref[...]
        h = dis * (agg_ref[0] + agg_ref[1] + y_ref[...]) + e0_ref[0]
        h = jnp.maximum(h, 0.0)
        out_ref[...] = dis * jnp.dot(
            h, w_ref[...], preferred_element_type=jnp.float32)

    return pl.pallas_call(
        body,
        grid=(n // br,),
        in_specs=[
            pl.BlockSpec((2, br, d), lambda i: (0, i, 0)),
            pl.BlockSpec((br, d), lambda i: (i, 0)),
            pl.BlockSpec((1, br, d), lambda i: (lidx, i, 0)),
            pl.BlockSpec((br, 1), lambda i: (i, 0)),
            pl.BlockSpec((d, d), lambda i: (0, 0)),
        ],
        out_specs=pl.BlockSpec((br, d), lambda i: (i, 0)),
        out_shape=jax.ShapeDtypeStruct((n, d), jnp.float32),
    )(agg2, y, e0, dis1, w_next)


def _tc_final(agg2, y, e0, dis1, lidx, n, d, br):
    def body(agg_ref, y_ref, e0_ref, dis_ref, out_ref):
        dis = dis_ref[...]
        out_ref[...] = dis * (agg_ref[0] + agg_ref[1] + y_ref[...]) + e0_ref[0]

    return pl.pallas_call(
        body,
        grid=(n // br,),
        in_specs=[
            pl.BlockSpec((2, br, d), lambda i: (0, i, 0)),
            pl.BlockSpec((br, d), lambda i: (i, 0)),
            pl.BlockSpec((1, br, d), lambda i: (lidx, i, 0)),
            pl.BlockSpec((br, 1), lambda i: (i, 0)),
        ],
        out_specs=pl.BlockSpec((br, d), lambda i: (i, 0)),
        out_shape=jax.ShapeDtypeStruct((n, d), jnp.float32),
    )(agg2, y, e0, dis1)


def kernel(x, edge_index, edge_attr,
           w1, b1, ew1, eb1,
           w2, b2, ew2, eb2,
           w3, b3, ew3, eb3,
           w4, b4, ew4, eb4,
           w5, b5, ew5, eb5):
    n, d = x.shape
    e = edge_index.shape[1]
    de = edge_attr.shape[1]
    nl = 5
    br = 2000

    epw = e // NW
    k = 100
    nch = epw // k

    rows3 = edge_index[0].reshape(NW, nch, k)
    cols3 = edge_index[1].reshape(NW, nch, k)
    ea3 = edge_attr.reshape(NW * nch, k, de)
    ka = 80
    packed3 = jnp.bitwise_or(
        jnp.left_shift(edge_index[1].astype(jnp.int32), 16),
        edge_index[0].astype(jnp.int32)).reshape(NW, epw // ka, ka)

    np_ = ((n + 8 * NS - 1) // (8 * NS)) * (8 * NS)
    rps = np_ // NS
    z128 = jnp.zeros((rps, d), jnp.float32)
    ones128 = jnp.concatenate(
        [jnp.ones((k, 1), jnp.float32), jnp.zeros((k, 127), jnp.float32)], axis=1)

    ws = [w1, w2, w3, w4, w5]
    ewts = jnp.stack([ew1.T, ew2.T, ew3.T, ew4.T, ew5.T])
    ebs3 = jnp.stack([eb1, eb2, eb3, eb4, eb5])[:, None, :]
    bs3 = jnp.stack([b1, b2, b3, b4, b5])[:, None, :]

    deg2 = _sc_degree(rows3, ones128, z128, np_).reshape(NC, np_, d)[:, :n, :16]
    dis1, dis128 = _tc_dis(deg2, n)
    sea2 = _sc_sea(rows3, cols3, ea3, dis128, z128, np_).reshape(NC, np_, d)[:, :n, :32]
    e0 = _tc_e0(sea2, dis1, ewts, ebs3, bs3, n, d, nl)

    y = _tc_y(x, ws[0], dis1, n, d, br)
    for l in range(nl):
        agg2 = _sc_agg(y, packed3, z128, n, d, np_).reshape(NC, np_, d)[:, :n]
        if l < nl - 1:
            y = _tc_fused(agg2, y, e0, dis1, ws[l + 1], l, n, d, br)
        else:
            return _tc_final(agg2, y, e0, dis1, l, n, d, br)


# speedup vs baseline: 16.0557x; 1.2928x over previous
import functools

import jax
import jax.numpy as jnp
from jax import lax
from jax.experimental import pallas as pl
from jax.experimental.pallas import tpu as pltpu
from jax.experimental.pallas import tpu_sc as plsc

NC = 2
NS = 16
NW = NC * NS


def _sc_mesh():
    return plsc.VectorSubcoreMesh(core_axis_name="c", subcore_axis_name="s")


def _sc_degree(packed3, ones128, z128, np_):
    nch, k = packed3.shape[1], packed3.shape[2]
    rps = np_ // NS

    @functools.partial(
        pl.kernel,
        mesh=_sc_mesh(),
        out_type=jax.ShapeDtypeStruct((NC * np_, 128), jnp.float32),
        scratch_types=[
            pltpu.VMEM_SHARED((np_, 128), jnp.float32),
            pltpu.VMEM((nch, k), jnp.int32),
            pltpu.VMEM((k,), jnp.int32),
            pltpu.VMEM((k, 128), jnp.float32),
        ],
    )
    def deg_kernel(packed_hbm, ones_hbm, z_hbm, out_hbm, acc_sh, pk_v,
                   ri_v, ones_v):
        cid = lax.axis_index("c")
        sid = lax.axis_index("s")
        wid = sid * NC + cid
        pltpu.sync_copy(z_hbm, acc_sh.at[pl.ds(sid * rps, rps)])
        pltpu.sync_copy(packed_hbm.at[wid], pk_v)
        pltpu.sync_copy(ones_hbm, ones_v)
        plsc.subcore_barrier()

        def chunk(i, c):
            for j in range(k // 16):
                v = pk_v[i, pl.ds(j * 16, 16)]
                ri_v[pl.ds(j * 16, 16)] = jnp.bitwise_and(v, 0xFFFF)
            pltpu.sync_copy(ones_v, acc_sh.at[ri_v], add=True)
            return c

        lax.fori_loop(0, nch, chunk, 0)
        plsc.subcore_barrier()
        pltpu.sync_copy(
            acc_sh.at[pl.ds(sid * rps, rps)],
            out_hbm.at[pl.ds(cid * np_ + sid * rps, rps)],
        )

    return deg_kernel(packed3, ones128, z128)


def _sc_sea(packed2, ea_a, dis128, z128, n, np_):
    epw = packed2.shape[1]
    de = ea_a.shape[2]
    k = ea_a.shape[1]
    nch = epw // k
    rps = np_ // NS
    NB = 2

    @functools.partial(
        pl.kernel,
        mesh=_sc_mesh(),
        out_type=jax.ShapeDtypeStruct((NC * np_, 128), jnp.float32),
        scratch_types=[
            pltpu.VMEM_SHARED((np_, 128), jnp.float32),
            pltpu.VMEM((epw,), jnp.int32),
            pltpu.VMEM((k,), jnp.int32),
            pltpu.VMEM((k,), jnp.int32),
            pltpu.VMEM((k,), jnp.int32),
            pltpu.VMEM((k,), jnp.int32),
            pltpu.VMEM((k, 128), jnp.float32),
            pltpu.VMEM((k, 128), jnp.float32),
            pltpu.VMEM((k, de), jnp.float32),
            pltpu.SemaphoreType.DMA,
            pltpu.SemaphoreType.DMA,
            pltpu.SemaphoreType.DMA,
        ],
    )
    def sea_kernel(packed_hbm, ea_hbm, dis_hbm, z_hbm, out_hbm, acc_sh,
                   pk_v, ri0, ci0, ri1, ci1, g0, g1, ea_v,
                   gsem0, gsem1, esem):
        cid = lax.axis_index("c")
        sid = lax.axis_index("s")
        wid = sid * NC + cid
        pltpu.sync_copy(z_hbm, acc_sh.at[pl.ds(sid * rps, rps)])
        pltpu.sync_copy(packed_hbm.at[wid], pk_v)
        plsc.subcore_barrier()

        ris = [ri0, ri1]
        cis = [ci0, ci1]
        gs = [g0, g1]
        gsems = [gsem0, gsem1]

        def unpack(i, b):
            for j in range(k // 16):
                v = pk_v[pl.ds(i * k + j * 16, 16)]
                ris[b][pl.ds(j * 16, 16)] = jnp.bitwise_and(v, 0xFFFF)
                cis[b][pl.ds(j * 16, 16)] = lax.shift_right_logical(v, 16)

        for b in range(NB):
            unpack(b, b)
            pltpu.async_copy(dis_hbm.at[ris[b]], gs[b], gsems[b])
        pltpu.async_copy(ea_hbm.at[wid * nch], ea_v, esem)

        ngrp = (nch + NB - 1) // NB

        def group(g, carry):
            for b in range(NB):
                i = g * NB + b

                @pl.when(i < nch)
                def _():
                    pltpu.make_async_copy(dis_hbm.at[ris[b]], gs[b],
                                          gsems[b]).wait()
                    pltpu.make_async_copy(ea_hbm.at[wid * nch + i], ea_v,
                                          esem).wait()
                    for e in range(k):
                        gs[b][e, 0:16] = gs[b][e, 0:16] * ea_v[e, 0:16]

                    @pl.when(i + 1 < nch)
                    def _():
                        pltpu.async_copy(ea_hbm.at[wid * nch + i + 1], ea_v,
                                         esem)

                    pltpu.sync_copy(gs[b], acc_sh.at[cis[b]], add=True)

                    @pl.when(i + NB < nch)
                    def _():
                        unpack(i + NB, b)
                        pltpu.async_copy(dis_hbm.at[ris[b]], gs[b], gsems[b])
            return carry

        lax.fori_loop(0, ngrp, group, 0)
        plsc.subcore_barrier()
        pltpu.sync_copy(
            acc_sh.at[pl.ds(sid * rps, rps)],
            out_hbm.at[pl.ds(cid * np_ + sid * rps, rps)],
        )

    return sea_kernel(packed2, ea_a, dis128, z128)


def _sc_agg(y, packed3, z128, n, d, np_):
    nch, k = packed3.shape[1], packed3.shape[2]
    rps = np_ // NS
    NB = 2

    @functools.partial(
        pl.kernel,
        mesh=_sc_mesh(),
        out_type=jax.ShapeDtypeStruct((NC * np_, d), jnp.float32),
        scratch_types=[
            pltpu.VMEM_SHARED((np_, d), jnp.float32),
            pltpu.VMEM((nch, k), jnp.int32),
            pltpu.VMEM((k,), jnp.int32),
            pltpu.VMEM((k,), jnp.int32),
            pltpu.VMEM((k,), jnp.int32),
            pltpu.VMEM((k,), jnp.int32),
            pltpu.VMEM((k, d), jnp.float32),
            pltpu.VMEM((k, d), jnp.float32),
            pltpu.SemaphoreType.DMA,
            pltpu.SemaphoreType.DMA,
        ],
    )
    def agg_kernel(y_hbm, packed_hbm, z_hbm, out_hbm, acc_sh, pk_v,
                   ri0, ci0, ri1, ci1, rows0, rows1, gsem0, gsem1):
        cid = lax.axis_index("c")
        sid = lax.axis_index("s")
        wid = sid * NC + cid
        pltpu.sync_copy(z_hbm, acc_sh.at[pl.ds(sid * rps, rps)])
        pltpu.sync_copy(packed_hbm.at[wid], pk_v)
        plsc.subcore_barrier()

        ris = [ri0, ri1]
        cis = [ci0, ci1]
        rows = [rows0, rows1]
        gsems = [gsem0, gsem1]

        def unpack(i, b):
            for j in range(k // 16):
                v = pk_v[i, pl.ds(j * 16, 16)]
                ris[b][pl.ds(j * 16, 16)] = jnp.bitwise_and(v, 0xFFFF)
                cis[b][pl.ds(j * 16, 16)] = lax.shift_right_logical(v, 16)

        for b in range(NB):
            unpack(b, b)
            pltpu.async_copy(y_hbm.at[ris[b]], rows[b], gsems[b])

        ngrp = (nch + NB - 1) // NB

        def group(g, carry):
            for b in range(NB):
                i = g * NB + b

                @pl.when(i < nch)
                def _():
                    pltpu.make_async_copy(y_hbm.at[ris[b]], rows[b],
                                          gsems[b]).wait()
                    pltpu.sync_copy(rows[b], acc_sh.at[cis[b]], add=True)

                    @pl.when(i + NB < nch)
                    def _():
                        unpack(i + NB, b)
                        pltpu.async_copy(y_hbm.at[ris[b]], rows[b], gsems[b])
            return carry

        lax.fori_loop(0, ngrp, group, 0)
        plsc.subcore_barrier()
        pltpu.sync_copy(
            acc_sh.at[pl.ds(sid * rps, rps)],
            out_hbm.at[pl.ds(cid * np_ + sid * rps, rps)],
        )

    return agg_kernel(y, packed3, z128)


def _tc_dis(deg2, n):
    def body(deg_ref, dis_ref, dis128_ref):
        d = deg_ref[0, :, 0:1] + deg_ref[1, :, 0:1] + 1.0
        dis = lax.rsqrt(d)
        dis_ref[...] = dis
        b16 = jnp.broadcast_to(dis, (n, 16))
        z111 = jnp.zeros((n, 111), jnp.float32)
        dis128_ref[...] = jnp.concatenate([b16, dis, z111], axis=1)

    return pl.pallas_call(
        body,
        out_shape=(
            jax.ShapeDtypeStruct((n, 1), jnp.float32),
            jax.ShapeDtypeStruct((n, 128), jnp.float32),
        ),
    )(deg2)


def _tc_e0(sea2, dis1, ewts, ebs3, bs3, n, d, nl):
    def body(sea_ref, dis_ref, ewt_ref, eb_ref, b_ref, out_ref):
        dis = dis_ref[...]
        sea = dis * (sea_ref[0, :, 0:16] + sea_ref[1, :, 0:16])
        ssum = dis * (sea_ref[0, :, 16:17] + sea_ref[1, :, 16:17]) + dis * dis
        term = jnp.dot(sea, ewt_ref[0], preferred_element_type=jnp.float32)
        out_ref[0] = term + ssum * eb_ref[0] + b_ref[0]

    return pl.pallas_call(
        body,
        grid=(nl,),
        in_specs=[
            pl.BlockSpec((2, n, 32), lambda l: (0, 0, 0)),
            pl.BlockSpec((n, 1), lambda l: (0, 0)),
            pl.BlockSpec((1, 16, d), lambda l: (l, 0, 0)),
            pl.BlockSpec((1, 1, d), lambda l: (l, 0, 0)),
            pl.BlockSpec((1, 1, d), lambda l: (l, 0, 0)),
        ],
        out_specs=pl.BlockSpec((1, n, d), lambda l: (l, 0, 0)),
        out_shape=jax.ShapeDtypeStruct((nl, n, d), jnp.float32),
    )(sea2, dis1, ewts, ebs3, bs3)


def _tc_y(h, w, dis1, n, d, br):
    def body(h_ref, w_ref, dis_ref, out_ref):
        out_ref[...] = dis_ref[...] * jnp.dot(
            h_ref[...], w_ref[...], preferred_element_type=jnp.float32)

    return pl.pallas_call(
        body,
        grid=(n // br,),
        in_specs=[
            pl.BlockSpec((br, d), lambda i: (i, 0)),
            pl.BlockSpec((d, d), lambda i: (0, 0)),
            pl.BlockSpec((br, 1), lambda i: (i, 0)),
        ],
        out_specs=pl.BlockSpec((br, d), lambda i: (i, 0)),
        out_shape=jax.ShapeDtypeStruct((n, d), jnp.float32),
    )(h, w, dis1)


def _tc_fused(agg2, y, e0, dis1, w_next, lidx, n, d, br):
    def body(agg_ref, y_ref, e0_ref, dis_ref, w_ref, out_ref):
        dis = dis_ref[...]
        h = dis * (agg_ref[0] + agg_ref[1] + y_ref[...]) + e0_ref[0]
        h = jnp.maximum(h, 0.0)
        out_ref[...] = dis * jnp.dot(
            h, w_ref[...], preferred_element_type=jnp.float32)

    return pl.pallas_call(
        body,
        grid=(n // br,),
        in_specs=[
            pl.BlockSpec((2, br, d), lambda i: (0, i, 0)),
            pl.BlockSpec((br, d), lambda i: (i, 0)),
            pl.BlockSpec((1, br, d), lambda i: (lidx, i, 0)),
            pl.BlockSpec((br, 1), lambda i: (i, 0)),
            pl.BlockSpec((d, d), lambda i: (0, 0)),
        ],
        out_specs=pl.BlockSpec((br, d), lambda i: (i, 0)),
        out_shape=jax.ShapeDtypeStruct((n, d), jnp.float32),
    )(agg2, y, e0, dis1, w_next)


def _tc_final(agg2, y, e0, dis1, lidx, n, d, br):
    def body(agg_ref, y_ref, e0_ref, dis_ref, out_ref):
        dis = dis_ref[...]
        out_ref[...] = dis * (agg_ref[0] + agg_ref[1] + y_ref[...]) + e0_ref[0]

    return pl.pallas_call(
        body,
        grid=(n // br,),
        in_specs=[
            pl.BlockSpec((2, br, d), lambda i: (0, i, 0)),
            pl.BlockSpec((br, d), lambda i: (i, 0)),
            pl.BlockSpec((1, br, d), lambda i: (lidx, i, 0)),
            pl.BlockSpec((br, 1), lambda i: (i, 0)),
        ],
        out_specs=pl.BlockSpec((br, d), lambda i: (i, 0)),
        out_shape=jax.ShapeDtypeStruct((n, d), jnp.float32),
    )(agg2, y, e0, dis1)


def kernel(x, edge_index, edge_attr,
           w1, b1, ew1, eb1,
           w2, b2, ew2, eb2,
           w3, b3, ew3, eb3,
           w4, b4, ew4, eb4,
           w5, b5, ew5, eb5):
    n, d = x.shape
    e = edge_index.shape[1]
    de = edge_attr.shape[1]
    nl = 5
    br = 2000

    epw = e // NW
    ka = 80
    nch = epw // ka
    packed3 = jnp.bitwise_or(
        jnp.left_shift(edge_index[1].astype(jnp.int32), 16),
        edge_index[0].astype(jnp.int32)).reshape(NW, nch, ka)
    ea_a = edge_attr.reshape(NW * nch, ka, de)

    np_ = ((n + 8 * NS - 1) // (8 * NS)) * (8 * NS)
    rps = np_ // NS
    z128 = jnp.zeros((rps, d), jnp.float32)

    ws = [w1, w2, w3, w4, w5]
    ewts = jnp.stack([ew1.T, ew2.T, ew3.T, ew4.T, ew5.T])
    ebs3 = jnp.stack([eb1, eb2, eb3, eb4, eb5])[:, None, :]
    bs3 = jnp.stack([b1, b2, b3, b4, b5])[:, None, :]

    ones128 = jnp.concatenate(
        [jnp.ones((ka, 1), jnp.float32), jnp.zeros((ka, 127), jnp.float32)],
        axis=1)
    deg2 = _sc_degree(packed3, ones128, z128, np_).reshape(NC, np_, d)[:, :n, :16]
    dis1, dis128 = _tc_dis(deg2, n)
    packed2 = packed3.reshape(NW, epw)
    sea2 = _sc_sea(packed2, ea_a, dis128, z128, n, np_).reshape(NC, np_, d)[:, :n, :32]
    e0 = _tc_e0(sea2, dis1, ewts, ebs3, bs3, n, d, nl)

    y = _tc_y(x, ws[0], dis1, n, d, br)
    for l in range(nl):
        agg2 = _sc_agg(y, packed3, z128, n, d, np_).reshape(NC, np_, d)[:, :n]
        if l < nl - 1:
            y = _tc_fused(agg2, y, e0, dis1, ws[l + 1], l, n, d, br)
        else:
            return _tc_final(agg2, y, e0, dis1, l, n, d, br)


# speedup vs baseline: 19.2640x; 1.1998x over previous
import functools

import jax
import jax.numpy as jnp
from jax import lax
from jax.experimental import pallas as pl
from jax.experimental.pallas import tpu as pltpu
from jax.experimental.pallas import tpu_sc as plsc

NC = 2
NS = 16
NW = NC * NS


def _sc_mesh():
    return plsc.VectorSubcoreMesh(core_axis_name="c", subcore_axis_name="s")


def _sc_degree(packed2, ones128, z128, np_):
    k = 80
    nch = packed2.shape[1] // k
    rps = np_ // NS

    @functools.partial(
        pl.kernel,
        mesh=_sc_mesh(),
        out_type=jax.ShapeDtypeStruct((NC * np_, 128), jnp.float32),
        scratch_types=[
            pltpu.VMEM_SHARED((np_, 128), jnp.float32),
            pltpu.VMEM((nch * k,), jnp.int32),
            pltpu.VMEM((k,), jnp.int32),
            pltpu.VMEM((k, 128), jnp.float32),
        ],
    )
    def deg_kernel(packed_hbm, ones_hbm, z_hbm, out_hbm, acc_sh, pk_v,
                   ri_v, ones_v):
        cid = lax.axis_index("c")
        sid = lax.axis_index("s")
        wid = sid * NC + cid
        pltpu.sync_copy(z_hbm, acc_sh.at[pl.ds(sid * rps, rps)])
        pltpu.sync_copy(packed_hbm.at[wid], pk_v)
        pltpu.sync_copy(ones_hbm, ones_v)
        plsc.subcore_barrier()

        def chunk(i, c):
            for j in range(k // 16):
                v = pk_v[pl.ds(i * k + j * 16, 16)]
                ri_v[pl.ds(j * 16, 16)] = jnp.bitwise_and(v, 0xFFFF)
            pltpu.sync_copy(ones_v, acc_sh.at[ri_v], add=True)
            return c

        lax.fori_loop(0, nch, chunk, 0)
        plsc.subcore_barrier()
        pltpu.sync_copy(
            acc_sh.at[pl.ds(sid * rps, rps)],
            out_hbm.at[pl.ds(cid * np_ + sid * rps, rps)],
        )

    return deg_kernel(packed2, ones128, z128)


def _sc_sea(packed2, ea_a, dis128, z128, n, np_):
    epw = packed2.shape[1]
    de = ea_a.shape[2]
    k = ea_a.shape[1]
    nch = epw // k
    rps = np_ // NS
    NB = 2

    @functools.partial(
        pl.kernel,
        mesh=_sc_mesh(),
        out_type=jax.ShapeDtypeStruct((NC * np_, 128), jnp.float32),
        scratch_types=[
            pltpu.VMEM_SHARED((np_, 128), jnp.float32),
            pltpu.VMEM((epw,), jnp.int32),
            pltpu.VMEM((k,), jnp.int32),
            pltpu.VMEM((k,), jnp.int32),
            pltpu.VMEM((k,), jnp.int32),
            pltpu.VMEM((k,), jnp.int32),
            pltpu.VMEM((k, 128), jnp.float32),
            pltpu.VMEM((k, 128), jnp.float32),
            pltpu.VMEM((k, de), jnp.float32),
            pltpu.SemaphoreType.DMA,
            pltpu.SemaphoreType.DMA,
            pltpu.SemaphoreType.DMA,
        ],
    )
    def sea_kernel(packed_hbm, ea_hbm, dis_hbm, z_hbm, out_hbm, acc_sh,
                   pk_v, ri0, ci0, ri1, ci1, g0, g1, ea_v,
                   gsem0, gsem1, esem):
        cid = lax.axis_index("c")
        sid = lax.axis_index("s")
        wid = sid * NC + cid
        pltpu.sync_copy(z_hbm, acc_sh.at[pl.ds(sid * rps, rps)])
        pltpu.sync_copy(packed_hbm.at[wid], pk_v)
        plsc.subcore_barrier()

        ris = [ri0, ri1]
        cis = [ci0, ci1]
        gs = [g0, g1]
        gsems = [gsem0, gsem1]

        def unpack(i, b):
            for j in range(k // 16):
                v = pk_v[pl.ds(i * k + j * 16, 16)]
                ris[b][pl.ds(j * 16, 16)] = jnp.bitwise_and(v, 0xFFFF)
                cis[b][pl.ds(j * 16, 16)] = lax.shift_right_logical(v, 16)

        for b in range(NB):
            unpack(b, b)
            pltpu.async_copy(dis_hbm.at[ris[b]], gs[b], gsems[b])
        pltpu.async_copy(ea_hbm.at[wid * nch], ea_v, esem)

        ngrp = (nch + NB - 1) // NB

        def group(g, carry):
            for b in range(NB):
                i = g * NB + b

                @pl.when(i < nch)
                def _():
                    pltpu.make_async_copy(dis_hbm.at[ris[b]], gs[b],
                                          gsems[b]).wait()
                    pltpu.make_async_copy(ea_hbm.at[wid * nch + i], ea_v,
                                          esem).wait()
                    for e in range(k):
                        gs[b][e, 0:16] = gs[b][e, 0:16] * ea_v[e, 0:16]

                    @pl.when(i + 1 < nch)
                    def _():
                        pltpu.async_copy(ea_hbm.at[wid * nch + i + 1], ea_v,
                                         esem)

                    pltpu.sync_copy(gs[b], acc_sh.at[cis[b]], add=True)

                    @pl.when(i + NB < nch)
                    def _():
                        unpack(i + NB, b)
                        pltpu.async_copy(dis_hbm.at[ris[b]], gs[b], gsems[b])
            return carry

        lax.fori_loop(0, ngrp, group, 0)
        plsc.subcore_barrier()
        pltpu.sync_copy(
            acc_sh.at[pl.ds(sid * rps, rps)],
            out_hbm.at[pl.ds(cid * np_ + sid * rps, rps)],
        )

    return sea_kernel(packed2, ea_a, dis128, z128)


def _sc_agg(y, packed2, z128, d, np_):
    epw = packed2.shape[1]
    k = 80
    nch = epw // k
    rps = np_ // NS
    NB = 3

    @functools.partial(
        pl.kernel,
        mesh=_sc_mesh(),
        out_type=jax.ShapeDtypeStruct((NC * np_, d), jnp.float32),
        scratch_types=[
            pltpu.VMEM_SHARED((np_, d), jnp.float32),
            pltpu.VMEM((epw,), jnp.int32),
            pltpu.VMEM((k,), jnp.int32),
            pltpu.VMEM((k,), jnp.int32),
            pltpu.VMEM((k,), jnp.int32),
            pltpu.VMEM((k,), jnp.int32),
            pltpu.VMEM((k,), jnp.int32),
            pltpu.VMEM((k,), jnp.int32),
            pltpu.VMEM((k, d), jnp.float32),
            pltpu.VMEM((k, d), jnp.float32),
            pltpu.VMEM((k, d), jnp.float32),
            pltpu.SemaphoreType.DMA,
            pltpu.SemaphoreType.DMA,
            pltpu.SemaphoreType.DMA,
        ],
    )
    def agg_kernel(y_hbm, packed_hbm, z_hbm, out_hbm, acc_sh, pk_v,
                   ri0, ci0, ri1, ci1, ri2, ci2, rows0, rows1, rows2,
                   gsem0, gsem1, gsem2):
        cid = lax.axis_index("c")
        sid = lax.axis_index("s")
        wid = sid * NC + cid
        pltpu.sync_copy(z_hbm, acc_sh.at[pl.ds(sid * rps, rps)])
        pltpu.sync_copy(packed_hbm.at[wid], pk_v)
        plsc.subcore_barrier()

        ris = [ri0, ri1, ri2]
        cis = [ci0, ci1, ci2]
        rows = [rows0, rows1, rows2]
        gsems = [gsem0, gsem1, gsem2]

        def unpack(i, b):
            for j in range(k // 16):
                v = pk_v[pl.ds(i * k + j * 16, 16)]
                ris[b][pl.ds(j * 16, 16)] = jnp.bitwise_and(v, 0xFFFF)
                cis[b][pl.ds(j * 16, 16)] = lax.shift_right_logical(v, 16)

        for b in range(NB):
            unpack(b, b)
            pltpu.async_copy(y_hbm.at[ris[b]], rows[b], gsems[b])

        ngrp = (nch + NB - 1) // NB

        def group(g, carry):
            for b in range(NB):
                i = g * NB + b

                @pl.when(i < nch)
                def _():
                    pltpu.make_async_copy(y_hbm.at[ris[b]], rows[b],
                                          gsems[b]).wait()
                    pltpu.sync_copy(rows[b], acc_sh.at[cis[b]], add=True)

                    @pl.when(i + NB < nch)
                    def _():
                        unpack(i + NB, b)
                        pltpu.async_copy(y_hbm.at[ris[b]], rows[b], gsems[b])
            return carry

        lax.fori_loop(0, ngrp, group, 0)
        plsc.subcore_barrier()
        pltpu.sync_copy(
            acc_sh.at[pl.ds(sid * rps, rps)],
            out_hbm.at[pl.ds(cid * np_ + sid * rps, rps)],
        )

    return agg_kernel(y, packed2, z128)


def _tc_dis(deg2, n):
    def body(deg_ref, dis_ref, dis128_ref):
        d = deg_ref[0, :, 0:1] + deg_ref[1, :, 0:1] + 1.0
        dis = lax.rsqrt(d)
        dis_ref[...] = dis
        b16 = jnp.broadcast_to(dis, (n, 16))
        z111 = jnp.zeros((n, 111), jnp.float32)
        dis128_ref[...] = jnp.concatenate([b16, dis, z111], axis=1)

    return pl.pallas_call(
        body,
        out_shape=(
            jax.ShapeDtypeStruct((n, 1), jnp.float32),
            jax.ShapeDtypeStruct((n, 128), jnp.float32),
        ),
    )(deg2)


def _tc_e0(sea2, dis1, ewts, ebs3, bs3, n, d, nl):
    def body(sea_ref, dis_ref, ewt_ref, eb_ref, b_ref, out_ref):
        dis = dis_ref[...]
        sea = dis * (sea_ref[0, :, 0:16] + sea_ref[1, :, 0:16])
        ssum = dis * (sea_ref[0, :, 16:17] + sea_ref[1, :, 16:17]) + dis * dis
        term = jnp.dot(sea, ewt_ref[0], preferred_element_type=jnp.float32)
        out_ref[0] = term + ssum * eb_ref[0] + b_ref[0]

    return pl.pallas_call(
        body,
        grid=(nl,),
        in_specs=[
            pl.BlockSpec((2, n, 128), lambda l: (0, 0, 0)),
            pl.BlockSpec((n, 1), lambda l: (0, 0)),
            pl.BlockSpec((1, 16, d), lambda l: (l, 0, 0)),
            pl.BlockSpec((1, 1, d), lambda l: (l, 0, 0)),
            pl.BlockSpec((1, 1, d), lambda l: (l, 0, 0)),
        ],
        out_specs=pl.BlockSpec((1, n, d), lambda l: (l, 0, 0)),
        out_shape=jax.ShapeDtypeStruct((nl, n, d), jnp.float32),
    )(sea2, dis1, ewts, ebs3, bs3)


def _tc_y(h, w, dis1, n, d, br):
    def body(h_ref, w_ref, dis_ref, out_ref):
        out_ref[...] = dis_ref[...] * jnp.dot(
            h_ref[...], w_ref[...], preferred_element_type=jnp.float32)

    return pl.pallas_call(
        body,
        grid=(n // br,),
        in_specs=[
            pl.BlockSpec((br, d), lambda i: (i, 0)),
            pl.BlockSpec((d, d), lambda i: (0, 0)),
            pl.BlockSpec((br, 1), lambda i: (i, 0)),
        ],
        out_specs=pl.BlockSpec((br, d), lambda i: (i, 0)),
        out_shape=jax.ShapeDtypeStruct((n, d), jnp.float32),
    )(h, w, dis1)


def _tc_fused(agg2, y, e0, dis1, w_next, lidx, n, d, br):
    def body(agg_ref, y_ref, e0_ref, dis_ref, w_ref, out_ref):
        dis = dis_ref[...]
        h = dis * (agg_ref[0] + agg_ref[1] + y_ref[...]) + e0_ref[0]
        h = jnp.maximum(h, 0.0)
        out_ref[...] = dis * jnp.dot(
            h, w_ref[...], preferred_element_type=jnp.float32)

    return pl.pallas_call(
        body,
        grid=(n // br,),
        in_specs=[
            pl.BlockSpec((2, br, d), lambda i: (0, i, 0)),
            pl.BlockSpec((br, d), lambda i: (i, 0)),
            pl.BlockSpec((1, br, d), lambda i: (lidx, i, 0)),
            pl.BlockSpec((br, 1), lambda i: (i, 0)),
            pl.BlockSpec((d, d), lambda i: (0, 0)),
        ],
        out_specs=pl.BlockSpec((br, d), lambda i: (i, 0)),
        out_shape=jax.ShapeDtypeStruct((n, d), jnp.float32),
    )(agg2, y, e0, dis1, w_next)


def _tc_final(agg2, y, e0, dis1, lidx, n, d, br):
    def body(agg_ref, y_ref, e0_ref, dis_ref, out_ref):
        dis = dis_ref[...]
        out_ref[...] = dis * (agg_ref[0] + agg_ref[1] + y_ref[...]) + e0_ref[0]

    return pl.pallas_call(
        body,
        grid=(n // br,),
        in_specs=[
            pl.BlockSpec((2, br, d), lambda i: (0, i, 0)),
            pl.BlockSpec((br, d), lambda i: (i, 0)),
            pl.BlockSpec((1, br, d), lambda i: (lidx, i, 0)),
            pl.BlockSpec((br, 1), lambda i: (i, 0)),
        ],
        out_specs=pl.BlockSpec((br, d), lambda i: (i, 0)),
        out_shape=jax.ShapeDtypeStruct((n, d), jnp.float32),
    )(agg2, y, e0, dis1)


def kernel(x, edge_index, edge_attr,
           w1, b1, ew1, eb1,
           w2, b2, ew2, eb2,
           w3, b3, ew3, eb3,
           w4, b4, ew4, eb4,
           w5, b5, ew5, eb5):
    n, d = x.shape
    e = edge_index.shape[1]
    de = edge_attr.shape[1]
    nl = 5
    br = 1280

    epw = e // NW
    ka = 80
    nch = epw // ka
    packed2 = jnp.bitwise_or(
        jnp.left_shift(edge_index[1].astype(jnp.int32), 16),
        edge_index[0].astype(jnp.int32)).reshape(NW, epw)
    ea_a = edge_attr.reshape(NW * nch, ka, de)

    np_ = ((n + 8 * NS - 1) // (8 * NS)) * (8 * NS)
    rps = np_ // NS
    z128 = jnp.zeros((rps, d), jnp.float32)

    ws = [w1, w2, w3, w4, w5]
    ewts = jnp.stack([ew1.T, ew2.T, ew3.T, ew4.T, ew5.T])
    ebs3 = jnp.stack([eb1, eb2, eb3, eb4, eb5])[:, None, :]
    bs3 = jnp.stack([b1, b2, b3, b4, b5])[:, None, :]

    ones128 = jnp.concatenate(
        [jnp.ones((ka, 1), jnp.float32), jnp.zeros((ka, 127), jnp.float32)],
        axis=1)
    deg2 = _sc_degree(packed2, ones128, z128, np_).reshape(NC, np_, d)
    dis1, dis128 = _tc_dis(deg2, np_)
    sea2 = _sc_sea(packed2, ea_a, dis128, z128, n, np_).reshape(NC, np_, d)
    e0 = _tc_e0(sea2, dis1, ewts, ebs3, bs3, np_, d, nl)

    x_pad = jnp.pad(x, ((0, np_ - n), (0, 0)))
    y = _tc_y(x_pad, ws[0], dis1, np_, d, br)
    for l in range(nl):
        agg2 = _sc_agg(y, packed2, z128, d, np_).reshape(NC, np_, d)
        if l < nl - 1:
            y = _tc_fused(agg2, y, e0, dis1, ws[l + 1], l, np_, d, br)
        else:
            return _tc_final(agg2, y, e0, dis1, l, np_, d, br)[:n]
